# Initial kernel scaffold; baseline (speedup 1.0000x reference)
import jax
import jax.numpy as jnp
from jax.experimental import pallas as pl


def kernel(x, weight, bias):
    raise NotImplementedError("write your pallas kernel here")


# speedup vs baseline: 2.1281x; 2.1281x over previous
import functools

import jax
import jax.numpy as jnp
from jax.experimental import pallas as pl
from jax.experimental.pallas import tpu as pltpu

_K = 4
_MIN_VALUE = 1.3862944
_MAX_VALUE = 1.4142135
_ROWS = 8


def _conv_clamp_kernel(w_ref, b_ref, x_ref, o_ref, q_ref, *, h, w, c_out):
    ho, wo = h - 1, w - 1
    x2 = x_ref[0]

    zrow = jnp.zeros((1, wo), jnp.float32)
    zcol = jnp.zeros((ho, 1), jnp.float32)
    for b in range(_K):
        q_ref[b, 0:1, :] = zrow
        q_ref[(_K - 1) * _K + b, ho - 1:ho, :] = zrow
    for a in range(_K):
        q_ref[a * _K, :, 0:1] = zcol
        q_ref[a * _K + _K - 1, :, wo - 1:wo] = zcol

    for a in range(_K):
        r0 = max(0, 1 - a)
        r1 = min(ho - 1, h - a)
        for b in range(_K):
            j0 = max(0, 1 - b)
            j1 = min(wo - 1, w - b)
            q_ref[a * _K + b, r0:r1 + 1, j0:j1 + 1] = (
                x2[r0 + a - 1:r1 + a, j0 + b - 1:j1 + b])

    wv = [[w_ref[c, t] for t in range(_K * _K)] for c in range(c_out)]
    bv = [b_ref[c] for c in range(c_out)]

    for rb in range(0, ho, _ROWS):
        r = min(_ROWS, ho - rb)
        taps = [q_ref[t, rb:rb + r, :] for t in range(_K * _K)]
        for c in range(c_out):
            acc = bv[c]
            for t in range(_K * _K):
                acc = acc + wv[c][t] * taps[t]
            o_ref[c, rb:rb + r, :] = jnp.clip(acc, _MIN_VALUE, _MAX_VALUE)


def kernel(x, weight, bias):
    n, cin, h, w = x.shape
    assert cin == 1 and weight.shape[0] == 1 and weight.shape[2:] == (_K, _K)
    c_out = weight.shape[1]
    ho, wo = h - 1, w - 1

    w_mat = weight[0, :, ::-1, ::-1].reshape(c_out, _K * _K).astype(jnp.float32)

    return pl.pallas_call(
        functools.partial(_conv_clamp_kernel, h=h, w=w, c_out=c_out),
        out_shape=jax.ShapeDtypeStruct((n, c_out, ho, wo), jnp.float32),
        grid=(n,),
        in_specs=[
            pl.BlockSpec(memory_space=pltpu.SMEM),
            pl.BlockSpec(memory_space=pltpu.SMEM),
            pl.BlockSpec((None, 1, h, w), lambda i: (i, 0, 0, 0)),
        ],
        out_specs=pl.BlockSpec((None, c_out, ho, wo), lambda i: (i, 0, 0, 0)),
        scratch_shapes=[pltpu.VMEM((_K * _K, ho, wo), jnp.float32)],
        compiler_params=pltpu.CompilerParams(
            dimension_semantics=("parallel",)),
    )(w_mat, bias.astype(jnp.float32), x)


# speedup vs baseline: 7.9751x; 3.7475x over previous
import functools

import jax
import jax.numpy as jnp
from jax.experimental import pallas as pl
from jax.experimental.pallas import tpu as pltpu

_K = 4
_MIN_VALUE = 1.3862944
_MAX_VALUE = 1.4142135
_J = 29


def _conv_clamp_kernel(l_ref, b_ref, t0, t1, t2, t3, o_ref, *, wo, c_out):
    n = t0.shape[-1]
    taps = (t0, t1, t2, t3)
    lmat = l_ref[...]
    bias = b_ref[...]
    j0s = list(range(0, wo - _J + 1, _J))
    if j0s[-1] != wo - _J:
        j0s.append(wo - _J)
    for j0 in j0s:
        slab = jnp.concatenate(
            [taps[a][0, j0 + 1:j0 + _J + 4, :] for a in range(_K)], axis=0)
        acc = jnp.dot(lmat, slab, preferred_element_type=jnp.float32) + bias
        acc = jnp.clip(acc, _MIN_VALUE, _MAX_VALUE)
        o_ref[0, j0:j0 + _J] = acc.reshape(_J, c_out, n)


def kernel(x, weight, bias):
    n, cin, h, w = x.shape
    assert cin == 1 and weight.shape[0] == 1 and weight.shape[2:] == (_K, _K)
    c_out = weight.shape[1]
    ho, wo = h - 1, w - 1

    wf = weight[0, :, ::-1, ::-1].astype(jnp.float32)

    rows = []
    for d in range(_J):
        band = jnp.pad(wf, ((0, 0), (0, 0), (d, _J - 1 - d)))
        rows.append(band.reshape(c_out, _K * (_J + 3)))
    lmat = jnp.concatenate(rows, axis=0)

    b_mat = jnp.tile(bias.astype(jnp.float32).reshape(1, c_out, 1),
                     (_J, 1, n)).reshape(_J * c_out, n)

    xt = jnp.pad(jnp.transpose(x[:, 0], (1, 2, 0)),
                 ((1, 1), (2, 1), (0, 0)))

    row_spec = [
        pl.BlockSpec((1, w + 3, n), (lambda a: (lambda i: (i + a, 0, 0)))(a))
        for a in range(_K)
    ]
    out_t = pl.pallas_call(
        functools.partial(_conv_clamp_kernel, wo=wo, c_out=c_out),
        out_shape=jax.ShapeDtypeStruct((ho, wo, c_out, n), jnp.float32),
        grid=(ho,),
        in_specs=[
            pl.BlockSpec((_J * c_out, _K * (_J + 3)), lambda i: (0, 0)),
            pl.BlockSpec((_J * c_out, n), lambda i: (0, 0)),
            *row_spec,
        ],
        out_specs=pl.BlockSpec((1, wo, c_out, n), lambda i: (i, 0, 0, 0)),
        compiler_params=pltpu.CompilerParams(
            dimension_semantics=("parallel",)),
    )(lmat, b_mat, xt, xt, xt, xt)

    return out_t.transpose(3, 2, 0, 1)


# speedup vs baseline: 11.4776x; 1.4392x over previous
import functools

import jax
import jax.numpy as jnp
from jax.experimental import pallas as pl
from jax.experimental.pallas import tpu as pltpu

_K = 4
_MIN_VALUE = 1.3862944
_MAX_VALUE = 1.4142135
_J = 29


_RI = 5


def _conv_clamp_kernel(l_ref, b_ref, tlo, thi, o_ref, *, wo, c_out):
    n = tlo.shape[-1]
    lmat = l_ref[...]
    bias = b_ref[...]
    j0s = list(range(0, wo - _J + 1, _J))
    if j0s[-1] != wo - _J:
        j0s.append(wo - _J)
    for r in range(_RI):
        taps = [tlo.at[r + a] if r + a < _RI else thi.at[r + a - _RI]
                for a in range(_K)]
        for j0 in j0s:
            slab = jnp.concatenate(
                [taps[a][j0 + 1:j0 + _J + 4, :] for a in range(_K)], axis=0)
            acc = jnp.dot(lmat, slab,
                          preferred_element_type=jnp.float32) + bias
            acc = jnp.clip(acc, _MIN_VALUE, _MAX_VALUE)
            o_ref[r, j0:j0 + _J] = acc.reshape(_J, c_out, n)


def kernel(x, weight, bias):
    n, cin, h, w = x.shape
    assert cin == 1 and weight.shape[0] == 1 and weight.shape[2:] == (_K, _K)
    c_out = weight.shape[1]
    ho, wo = h - 1, w - 1

    wf = weight[0, :, ::-1, ::-1].astype(jnp.float32)

    rows = []
    for d in range(_J):
        band = jnp.pad(wf, ((0, 0), (0, 0), (d, _J - 1 - d)))
        rows.append(band.reshape(c_out, _K * (_J + 3)))
    lmat = jnp.concatenate(rows, axis=0)

    b_mat = jnp.tile(bias.astype(jnp.float32).reshape(1, c_out, 1),
                     (_J, 1, n)).reshape(_J * c_out, n)

    assert ho % _RI == 0
    xt = jnp.pad(jnp.transpose(x[:, 0], (1, 2, 0)),
                 ((1, _RI - 2), (2, 1), (0, 0)))

    out_t = pl.pallas_call(
        functools.partial(_conv_clamp_kernel, wo=wo, c_out=c_out),
        out_shape=jax.ShapeDtypeStruct((ho, wo, c_out, n), jnp.float32),
        grid=(ho // _RI,),
        in_specs=[
            pl.BlockSpec((_J * c_out, _K * (_J + 3)), lambda k: (0, 0)),
            pl.BlockSpec((_J * c_out, n), lambda k: (0, 0)),
            pl.BlockSpec((_RI, w + 3, n), lambda k: (k, 0, 0)),
            pl.BlockSpec((_RI, w + 3, n), lambda k: (k + 1, 0, 0)),
        ],
        out_specs=pl.BlockSpec((_RI, wo, c_out, n),
                               lambda k: (k, 0, 0, 0)),
        compiler_params=pltpu.CompilerParams(
            dimension_semantics=("parallel",)),
    )(lmat, b_mat, xt, xt)

    return out_t.transpose(3, 2, 0, 1)


# speedup vs baseline: 11.5314x; 1.0047x over previous
import functools

import jax
import jax.numpy as jnp
from jax.experimental import pallas as pl
from jax.experimental.pallas import tpu as pltpu

_K = 4
_MIN_VALUE = 1.3862944
_MAX_VALUE = 1.4142135
_J = 32
_SEG = 40
_RI = 5


def _conv_clamp_kernel(l_ref, tlo, thi, o_ref, *, wo, c_out):
    n = tlo.shape[-1]
    lmat = l_ref[...]
    ones = jnp.ones((1, n), jnp.float32)
    j0s = list(range(0, wo - _J + 1, _J))
    if j0s[-1] != wo - _J:
        j0s.append(wo - _J)
    for r in range(_RI):
        taps = [tlo.at[r + a] if r + a < _RI else thi.at[r + a - _RI]
                for a in range(_K)]
        for j0 in j0s:
            slab = jnp.concatenate(
                [taps[a][j0:j0 + _SEG, :] for a in range(_K)] + [ones],
                axis=0)
            acc = jnp.dot(lmat, slab, preferred_element_type=jnp.float32)
            acc = jnp.clip(acc, _MIN_VALUE, _MAX_VALUE)
            o_ref[r, j0:j0 + _J] = acc.reshape(_J, c_out, n)


def kernel(x, weight, bias):
    n, cin, h, w = x.shape
    assert cin == 1 and weight.shape[0] == 1 and weight.shape[2:] == (_K, _K)
    c_out = weight.shape[1]
    ho, wo = h - 1, w - 1

    wf = weight[0, :, ::-1, ::-1].astype(jnp.float32)

    rows = []
    for d in range(_J):
        band = jnp.pad(wf, ((0, 0), (0, 0), (d, _SEG - 4 - d)))
        rows.append(band.reshape(c_out, _K * _SEG))
    lmat = jnp.concatenate(rows, axis=0)
    b_col = jnp.tile(bias.astype(jnp.float32).reshape(1, c_out),
                     (_J, 1)).reshape(_J * c_out, 1)
    lmat = jnp.concatenate([lmat, b_col], axis=1)

    assert ho % _RI == 0
    xt = jnp.pad(jnp.transpose(x[:, 0], (1, 2, 0)),
                 ((1, _RI - 2), (1, _SEG - _J - 1), (0, 0)))

    out_t = pl.pallas_call(
        functools.partial(_conv_clamp_kernel, wo=wo, c_out=c_out),
        out_shape=jax.ShapeDtypeStruct((ho, wo, c_out, n), jnp.float32),
        grid=(ho // _RI,),
        in_specs=[
            pl.BlockSpec((_J * c_out, _K * _SEG + 1), lambda k: (0, 0)),
            pl.BlockSpec((_RI, w + _SEG - _J, n), lambda k: (k, 0, 0)),
            pl.BlockSpec((_RI, w + _SEG - _J, n), lambda k: (k + 1, 0, 0)),
        ],
        out_specs=pl.BlockSpec((_RI, wo, c_out, n),
                               lambda k: (k, 0, 0, 0)),
        compiler_params=pltpu.CompilerParams(
            dimension_semantics=("parallel",)),
    )(lmat, xt, xt)

    return out_t.transpose(3, 2, 0, 1)


# speedup vs baseline: 14.2387x; 1.2348x over previous
import functools

import jax
import jax.numpy as jnp
from jax.experimental import pallas as pl
from jax.experimental.pallas import tpu as pltpu

_K = 4
_MIN_VALUE = 1.3862944
_MAX_VALUE = 1.4142135
_J = 32
_SEG = 40
_RI = 5
_NR = _RI + _K - 1


def _conv_clamp_kernel(l_ref, *refs, wo, c_out, h):
    rows, o_ref, s_ref = refs[:_NR], refs[_NR], refs[_NR + 1]
    n = rows[0].shape[0]
    w = rows[0].shape[3]
    pid = pl.program_id(0)
    lmat = l_ref[...]
    ones = jnp.ones((1, n), jnp.float32)

    s_ref[:, 0:1, :] = jnp.zeros((_NR, 1, n), jnp.float32)
    s_ref[:, w + 1:, :] = jnp.zeros((_NR, _SEG - _J - 1, n), jnp.float32)
    for t in range(_NR):
        vt = jnp.transpose(rows[t][:, 0, 0, :], (1, 0))
        if t == 0:
            vt = jnp.where(pid > 0, vt, 0.0)
        if t == _NR - 1:
            vt = jnp.where(_RI * pid - 1 + t <= h - 1, vt, 0.0)
        s_ref[t, 1:w + 1, :] = vt

    j0s = list(range(0, wo - _J + 1, _J))
    if j0s[-1] != wo - _J:
        j0s.append(wo - _J)
    for r in range(_RI):
        for j0 in j0s:
            slab = jnp.concatenate(
                [s_ref[r + a, j0:j0 + _SEG, :] for a in range(_K)] + [ones],
                axis=0)
            acc = jnp.dot(lmat, slab, preferred_element_type=jnp.float32)
            acc = jnp.clip(acc, _MIN_VALUE, _MAX_VALUE)
            o_ref[r, j0:j0 + _J] = acc.reshape(_J, c_out, n)


def kernel(x, weight, bias):
    n, cin, h, w = x.shape
    assert cin == 1 and weight.shape[0] == 1 and weight.shape[2:] == (_K, _K)
    c_out = weight.shape[1]
    ho, wo = h - 1, w - 1
    assert ho % _RI == 0

    wf = weight[0, :, ::-1, ::-1].astype(jnp.float32)

    rows = []
    for d in range(_J):
        band = jnp.pad(wf, ((0, 0), (0, 0), (d, _SEG - 4 - d)))
        rows.append(band.reshape(c_out, _K * _SEG))
    lmat = jnp.concatenate(rows, axis=0)
    b_col = jnp.tile(bias.astype(jnp.float32).reshape(1, c_out),
                     (_J, 1)).reshape(_J * c_out, 1)
    lmat = jnp.concatenate([lmat, b_col], axis=1)

    x3 = x.reshape(n, h, 1, w)

    def row_map(t):
        return lambda k: (0, jnp.clip(_RI * k - 1 + t, 0, h - 1), 0, 0)

    out_t = pl.pallas_call(
        functools.partial(_conv_clamp_kernel, wo=wo, c_out=c_out, h=h),
        out_shape=jax.ShapeDtypeStruct((ho, wo, c_out, n), jnp.float32),
        grid=(ho // _RI,),
        in_specs=[
            pl.BlockSpec((_J * c_out, _K * _SEG + 1), lambda k: (0, 0)),
            *[pl.BlockSpec((n, 1, 1, w), row_map(t)) for t in range(_NR)],
        ],
        out_specs=pl.BlockSpec((_RI, wo, c_out, n),
                               lambda k: (k, 0, 0, 0)),
        scratch_shapes=[pltpu.VMEM((_NR, w + _SEG - _J, n), jnp.float32)],
        compiler_params=pltpu.CompilerParams(
            dimension_semantics=("parallel",)),
    )(lmat, *([x3] * _NR))

    return out_t.transpose(3, 2, 0, 1)
